# Initial kernel scaffold; baseline (speedup 1.0000x reference)
import jax
import jax.numpy as jnp
from jax.experimental import pallas as pl


def kernel(cls_p3, cls_p4, cls_p5, cls_p6, cls_p7, cen_p3, cen_p4, cen_p5, cen_p6, cen_p7, reg_p3, reg_p4, reg_p5, reg_p6, reg_p7, gt_box, labels):
    raise NotImplementedError("write your pallas kernel here")


# speedup vs baseline: 4.0089x; 4.0089x over previous
import functools

import numpy as np
import jax
import jax.numpy as jnp
from jax import lax
from jax.experimental import pallas as pl

_STRIDES = (8, 16, 32, 64, 128)
_LIMITS = ((-1.0, 64.0), (64.0, 128.0), (128.0, 256.0), (256.0, 512.0),
           (512.0, 999999.0))
_SAMPLE_RADIO_RATIO = 1.5
_HWS = ((64, 64), (32, 32), (16, 16), (8, 8), (4, 4))
_P = sum(h * w for h, w in _HWS)
_T = 512
_NT = -(-_P // _T)
_P_PAD = _NT * _T


def _point_table() -> np.ndarray:
    xs, ys, lo, hi, ra = [], [], [], [], []
    for (h, w), s, (l0, l1) in zip(_HWS, _STRIDES, _LIMITS):
        gx = np.arange(w, dtype=np.float32) * s + s // 2
        gy = np.arange(h, dtype=np.float32) * s + s // 2
        yy, xx = np.meshgrid(gy, gx, indexing="ij")
        xs.append(xx.reshape(-1))
        ys.append(yy.reshape(-1))
        lo.append(np.full(h * w, l0, np.float32))
        hi.append(np.full(h * w, l1, np.float32))
        ra.append(np.full(h * w, s * _SAMPLE_RADIO_RATIO, np.float32))
    out = np.zeros((8, _P_PAD), np.float32)
    for row, vals in enumerate((xs, ys, lo, hi, ra)):
        v = np.concatenate(vals)
        out[row, :_P] = v
    return out


_PTS = _point_table()


def _body(pts_ref, gt_ref, cls_ref, cen_ref, reg_ref):
    x = pts_ref[0:1, :]
    y = pts_ref[1:2, :]
    lo = pts_ref[2:3, :]
    hi = pts_ref[3:4, :]
    ratio = pts_ref[4:5, :]
    g = gt_ref[0]
    x1 = g[:, 0:1]
    y1 = g[:, 1:2]
    x2 = g[:, 2:3]
    y2 = g[:, 3:4]
    lab = g[:, 4:5]
    m = g.shape[0]
    t = x.shape[1]

    l_ = x - x1
    t_ = y - y1
    r_ = x2 - x
    b_ = y2 - y
    off_min = jnp.minimum(jnp.minimum(l_, t_), jnp.minimum(r_, b_))
    off_max = jnp.maximum(jnp.maximum(l_, t_), jnp.maximum(r_, b_))
    area = (l_ + r_) * (t_ + b_)
    mask_gt = off_min > 0.0
    mask_lv = (off_max > lo) & (off_max <= hi)
    cx = (x1 + x2) / 2.0
    cy = (y1 + y2) / 2.0
    gmax = jnp.maximum(jnp.maximum(x - cx, cx - x),
                       jnp.maximum(y - cy, cy - y))
    mask_c = gmax < ratio
    pos = mask_gt & mask_lv & mask_c
    area_m = jnp.where(pos, area, 99999999.0)
    mn = jnp.min(area_m, axis=0, keepdims=True)
    ii = lax.broadcasted_iota(jnp.int32, (m, t), 0)
    sel = jnp.min(jnp.where(area_m == mn, ii, m), axis=0, keepdims=True)
    onehot = (ii == sel).astype(jnp.float32)
    rl = jnp.sum(l_ * onehot, axis=0, keepdims=True)
    rt = jnp.sum(t_ * onehot, axis=0, keepdims=True)
    rr = jnp.sum(r_ * onehot, axis=0, keepdims=True)
    rb = jnp.sum(b_ * onehot, axis=0, keepdims=True)
    cls = jnp.sum(lab * onehot, axis=0, keepdims=True)
    npos = jnp.sum(pos.astype(jnp.float32), axis=0, keepdims=True)
    pos2 = npos >= 1.0
    lr_min = jnp.minimum(rl, rr)
    lr_max = jnp.maximum(rl, rr)
    tb_min = jnp.minimum(rt, rb)
    tb_max = jnp.maximum(rt, rb)
    val = lr_min * tb_min / (lr_max * tb_max + 1e-10)
    cen = jnp.where(pos2, jnp.sqrt(jnp.where(pos2, val, 1.0)), -1.0)
    cls_ref[0] = jnp.where(pos2, cls, 0.0)
    cen_ref[0] = cen
    reg_ref[0] = jnp.concatenate(
        [jnp.where(pos2, rl, -1.0), jnp.where(pos2, rt, -1.0),
         jnp.where(pos2, rr, -1.0), jnp.where(pos2, rb, -1.0)], axis=0)


@functools.partial(jax.jit, static_argnames=("interpret",))
def _gen_targets(gt_box, labels, interpret=False):
    bsz, m = labels.shape
    pts = jnp.asarray(_PTS)
    gtp = jnp.concatenate(
        [gt_box.astype(jnp.float32),
         labels.astype(jnp.float32)[..., None],
         jnp.zeros((bsz, m, 3), jnp.float32)], axis=-1)
    o_cls, o_cen, o_reg = pl.pallas_call(
        _body,
        grid=(bsz, _NT),
        in_specs=[
            pl.BlockSpec((8, _T), lambda b, i: (0, i)),
            pl.BlockSpec((1, m, 8), lambda b, i: (b, 0, 0)),
        ],
        out_specs=[
            pl.BlockSpec((1, 1, _T), lambda b, i: (b, 0, i)),
            pl.BlockSpec((1, 1, _T), lambda b, i: (b, 0, i)),
            pl.BlockSpec((1, 4, _T), lambda b, i: (b, 0, i)),
        ],
        out_shape=[
            jax.ShapeDtypeStruct((bsz, 1, _P_PAD), jnp.float32),
            jax.ShapeDtypeStruct((bsz, 1, _P_PAD), jnp.float32),
            jax.ShapeDtypeStruct((bsz, 4, _P_PAD), jnp.float32),
        ],
        interpret=interpret,
    )(pts, gtp)
    cls_t = o_cls[:, 0, :_P, None].astype(jnp.int32)
    cen_t = o_cen[:, 0, :_P, None]
    reg_t = o_reg[:, :, :_P].transpose(0, 2, 1)
    return cls_t, cen_t, reg_t


def kernel(cls_p3, cls_p4, cls_p5, cls_p6, cls_p7,
           cen_p3, cen_p4, cen_p5, cen_p6, cen_p7,
           reg_p3, reg_p4, reg_p5, reg_p6, reg_p7,
           gt_box, labels):
    return _gen_targets(gt_box, labels)


# speedup vs baseline: 6.1762x; 1.5406x over previous
import functools

import numpy as np
import jax
import jax.numpy as jnp
from jax import lax
from jax.experimental import pallas as pl
from jax.experimental.pallas import tpu as pltpu
from jax.experimental.pallas import tpu_sc as plsc

_STRIDES = (8, 16, 32, 64, 128)
_LIMITS = ((-1.0, 64.0), (64.0, 128.0), (128.0, 256.0), (256.0, 512.0),
           (512.0, 999999.0))
_SAMPLE_RADIO_RATIO = 1.5
_HWS = ((64, 64), (32, 32), (16, 16), (8, 8), (4, 4))
_P = sum(h * w for h, w in _HWS)
_PSC = 5504
_CH = 1376
_BIG = 99999999.0
_LV = tuple(
    (sum(h * w for h, w in _HWS[:i]), _HWS[i][1], _HWS[i][0],
     float(_STRIDES[i]), _LIMITS[i][0], _LIMITS[i][1])
    for i in range(5))


def _point_xy() -> np.ndarray:
    xs, ys = [], []
    for (h, w), s in zip(_HWS, _STRIDES):
        gx = np.arange(w, dtype=np.float32) * s + s // 2
        gy = np.arange(h, dtype=np.float32) * s + s // 2
        yy, xx = np.meshgrid(gy, gx, indexing="ij")
        xs.append(xx.reshape(-1))
        ys.append(yy.reshape(-1))
    out = np.zeros((2, _PSC), np.float32)
    out[0, :_P] = np.concatenate(xs)
    out[1, :_P] = np.concatenate(ys)
    return out


_PTS_XY = _point_xy()


def _sc_body(gt_hbm, ptx_hbm, pty_hbm,
             o_cls, o_cen, o_rl, o_rt, o_rr, o_rb,
             ba, bi, gv, mb_a, mb_i, xyv, ocv, sh_a, sh_i):
    c = lax.axis_index("c")
    s = lax.axis_index("s")
    b = c * 4 + s % 4
    g = s // 4
    lane = lax.broadcasted_iota(jnp.int32, (16,), 0)
    zf = jnp.zeros((16,), jnp.float32)
    zi = jnp.zeros((16,), jnp.int32)

    pltpu.sync_copy(gt_hbm.at[pl.ds(b * 512, 512)], gv)

    def init_body(i, carry):
        ba[pl.ds(i * 16, 16)] = zf + _BIG
        bi[pl.ds(i * 16, 16)] = zf
        return carry
    lax.fori_loop(0, _PSC // 16, init_body, 0)

    dx = lane & 3
    dy = lane >> 2

    def box_body(j, carry):
        m = g * 16 + j
        col = zi + m
        x1 = plsc.load_gather(gv, [col])
        y1 = plsc.load_gather(gv, [col + 64])
        x2 = plsc.load_gather(gv, [col + 128])
        y2 = plsc.load_gather(gv, [col + 192])
        cx = (x1 + x2) / 2.0
        cy = (y1 + y2) / 2.0
        mf = zf + m.astype(jnp.float32)
        for off0, w, h, st, lo, hi in _LV:
            bx = (cx * (1.0 / st)).astype(jnp.int32) - 1
            by = (cy * (1.0 / st)).astype(jnp.int32) - 1
            ix = bx + dx
            iy = by + dy
            inb = (ix >= 0) & (ix < w) & (iy >= 0) & (iy < h)
            x = ix.astype(jnp.float32) * st + float(int(st) // 2)
            y = iy.astype(jnp.float32) * st + float(int(st) // 2)
            l_ = x - x1
            t_ = y - y1
            r_ = x2 - x
            b_ = y2 - y
            omin = jnp.minimum(jnp.minimum(l_, t_), jnp.minimum(r_, b_))
            omax = jnp.maximum(jnp.maximum(l_, t_), jnp.maximum(r_, b_))
            area = (l_ + r_) * (t_ + b_)
            gmax = jnp.maximum(jnp.maximum(x - cx, cx - x),
                               jnp.maximum(y - cy, cy - y))
            pos = ((omin > 0.0) & (omax > lo) & (omax <= hi)
                   & (gmax < st * _SAMPLE_RADIO_RATIO) & inb)
            p = off0 + iy * w + ix
            p = jnp.clip(p, 0, _PSC - 1)
            cur = plsc.load_gather(ba, [p])
            better = pos & (area < cur)
            plsc.store_scatter(ba, [p], area, mask=better)
            plsc.store_scatter(bi, [p], mf, mask=better)
        return carry
    lax.fori_loop(0, 16, box_body, 0)

    pltpu.sync_copy(ba, sh_a.at[pl.ds(s * _PSC, _PSC)])
    pltpu.sync_copy(bi, sh_i.at[pl.ds(s * _PSC, _PSC)])
    plsc.subcore_barrier()

    base = g * _CH
    for gg in range(4):
        spub = gg * 4 + s % 4
        pltpu.sync_copy(sh_a.at[pl.ds(spub * _PSC + base, _CH)],
                        mb_a.at[pl.ds(gg * _CH, _CH)])
        pltpu.sync_copy(sh_i.at[pl.ds(spub * _PSC + base, _CH)],
                        mb_i.at[pl.ds(gg * _CH, _CH)])
    pltpu.sync_copy(ptx_hbm.at[pl.ds(base, _CH)], xyv.at[pl.ds(0, _CH)])
    pltpu.sync_copy(pty_hbm.at[pl.ds(base, _CH)], xyv.at[pl.ds(_CH, _CH)])

    def fin_body(i, carry):
        o16 = i * 16
        best_a = mb_a[pl.ds(o16, 16)]
        best_i = mb_i[pl.ds(o16, 16)]
        for gg in range(1, 4):
            ag = mb_a[pl.ds(gg * _CH + o16, 16)]
            take = ag < best_a
            best_a = jnp.where(take, ag, best_a)
            best_i = jnp.where(take, mb_i[pl.ds(gg * _CH + o16, 16)], best_i)
        pos2 = best_a < _BIG
        idxv = best_i.astype(jnp.int32)
        x = xyv[pl.ds(o16, 16)]
        y = xyv[pl.ds(_CH + o16, 16)]
        x1 = plsc.load_gather(gv, [idxv])
        y1 = plsc.load_gather(gv, [idxv + 64])
        x2 = plsc.load_gather(gv, [idxv + 128])
        y2 = plsc.load_gather(gv, [idxv + 192])
        lab = plsc.load_gather(gv, [idxv + 256])
        l_ = x - x1
        t_ = y - y1
        r_ = x2 - x
        b_ = y2 - y
        lr_min = jnp.minimum(l_, r_)
        lr_max = jnp.maximum(l_, r_)
        tb_min = jnp.minimum(t_, b_)
        tb_max = jnp.maximum(t_, b_)
        val = lr_min * tb_min / (lr_max * tb_max + 1e-10)
        v = jnp.maximum(jnp.where(pos2, val, 1.0), 1e-30)
        ry = lax.bitcast_convert_type(
            0x5F3759DF - lax.shift_right_logical(
                lax.bitcast_convert_type(v, jnp.int32), 1), jnp.float32)
        for _ in range(3):
            ry = ry * (1.5 - 0.5 * v * ry * ry)
        sq = v * ry
        sq = 0.5 * (sq + v / sq)
        ocv[pl.ds(o16, 16)] = jnp.where(pos2, lab, 0.0)
        ocv[pl.ds(_CH + o16, 16)] = jnp.where(pos2, sq, -1.0)
        ocv[pl.ds(2 * _CH + o16, 16)] = jnp.where(pos2, l_, -1.0)
        ocv[pl.ds(3 * _CH + o16, 16)] = jnp.where(pos2, t_, -1.0)
        ocv[pl.ds(4 * _CH + o16, 16)] = jnp.where(pos2, r_, -1.0)
        ocv[pl.ds(5 * _CH + o16, 16)] = jnp.where(pos2, b_, -1.0)
        return carry
    lax.fori_loop(0, _CH // 16, fin_body, 0)

    obase = b * _PSC + base
    for r, oref in enumerate((o_cls, o_cen, o_rl, o_rt, o_rr, o_rb)):
        pltpu.sync_copy(ocv.at[pl.ds(r * _CH, _CH)], oref.at[pl.ds(obase, _CH)])


@jax.jit
def _gen_targets_sc(gt_box, labels):
    bsz, m = labels.shape
    gtp = jnp.concatenate(
        [gt_box.astype(jnp.float32).transpose(0, 2, 1),
         labels.astype(jnp.float32)[:, None, :],
         jnp.zeros((bsz, 3, m), jnp.float32)],
        axis=1).reshape(bsz * 8 * m)
    pts = jnp.asarray(_PTS_XY)
    f32 = jnp.float32
    sc_fn = pl.kernel(
        _sc_body,
        mesh=plsc.VectorSubcoreMesh(core_axis_name="c", subcore_axis_name="s"),
        compiler_params=pltpu.CompilerParams(needs_layout_passes=False),
        out_type=[jax.ShapeDtypeStruct((bsz * _PSC,), f32) for _ in range(6)],
        scratch_types=[
            pltpu.VMEM((_PSC,), f32),
            pltpu.VMEM((_PSC,), f32),
            pltpu.VMEM((512,), f32),
            pltpu.VMEM((4 * _CH,), f32),
            pltpu.VMEM((4 * _CH,), f32),
            pltpu.VMEM((2 * _CH,), f32),
            pltpu.VMEM((6 * _CH,), f32),
            pltpu.VMEM_SHARED((16 * _PSC,), f32),
            pltpu.VMEM_SHARED((16 * _PSC,), f32),
        ],
    )
    outs = sc_fn(gtp, pts[0], pts[1])
    o_cls, o_cen, o_rl, o_rt, o_rr, o_rb = (
        o.reshape(bsz, _PSC)[:, :_P] for o in outs)
    cls_t = o_cls[..., None].astype(jnp.int32)
    cen_t = o_cen[..., None]
    reg_t = jnp.stack([o_rl, o_rt, o_rr, o_rb], axis=-1)
    return cls_t, cen_t, reg_t


def kernel(cls_p3, cls_p4, cls_p5, cls_p6, cls_p7,
           cen_p3, cen_p4, cen_p5, cen_p6, cen_p7,
           reg_p3, reg_p4, reg_p5, reg_p6, reg_p7,
           gt_box, labels):
    return _gen_targets_sc(gt_box, labels)
